# baseline (speedup 1.0000x reference)
import functools

import jax
import jax.numpy as jnp
from jax import lax
from jax.experimental import pallas as pl
from jax.experimental.pallas import tpu as pltpu
from jax.experimental.pallas import tpu_sc as plsc

BATCH = 16384
EMB = 32
EPS = 1e-5

NC = 2
NS = 16
NW = NC * NS
BPW = BATCH // NW
CHUNK = 128
NCHUNK = BPW // CHUNK


def _sc_gather_body(uidx, iidx, Ug, Ig, Um, Im,
                    out_ug, out_ig, out_um, out_im,
                    idx_u, idx_i, sem):
    wid = lax.axis_index("s") * NC + lax.axis_index("c")
    base = wid * BPW
    pltpu.sync_copy(uidx.at[pl.ds(base, BPW)], idx_u)
    pltpu.sync_copy(iidx.at[pl.ds(base, BPW)], idx_i)

    def group(g, _):
        off = g * 16
        uvec = idx_u[pl.ds(off, 16)]
        ivec = idx_i[pl.ds(off, 16)]
        for k in range(16):
            u = uvec[k]
            t = ivec[k]
            r = base + off + k
            pltpu.make_async_copy(Ug.at[u], out_ug.at[r], sem).start()
            pltpu.make_async_copy(Um.at[u], out_um.at[r], sem).start()
            pltpu.make_async_copy(Ig.at[t], out_ig.at[r], sem).start()
            pltpu.make_async_copy(Im.at[t], out_im.at[r], sem).start()
        return ()

    lax.fori_loop(0, BPW // 16, group, (), unroll=False)
    dst = pl.ds(base, BPW)
    pltpu.make_async_copy(Ug.at[pl.ds(0, BPW)], out_ug.at[dst], sem).wait()
    pltpu.make_async_copy(Um.at[pl.ds(0, BPW)], out_um.at[dst], sem).wait()
    pltpu.make_async_copy(Ig.at[pl.ds(0, BPW)], out_ig.at[dst], sem).wait()
    pltpu.make_async_copy(Im.at[pl.ds(0, BPW)], out_im.at[dst], sem).wait()


def _sc_gather(uidx, iidx, Ug, Ig, Um, Im):
    mesh = plsc.VectorSubcoreMesh(core_axis_name="c", subcore_axis_name="s",
                                  num_cores=NC, num_subcores=NS)
    row = jax.ShapeDtypeStruct((BATCH, EMB), jnp.float32)
    f = pl.kernel(
        _sc_gather_body,
        out_type=(row, row, row, row),
        mesh=mesh,
        scratch_types=[
            pltpu.VMEM((BPW,), jnp.int32),
            pltpu.VMEM((BPW,), jnp.int32),
            pltpu.SemaphoreType.DMA,
        ],
    )
    return f(uidx, iidx, Ug, Ig, Um, Im)


def _tc_tail_body(ug_ref, ig_ref, um_ref, im_ref, w1_ref, b1_ref, g1_ref,
                  be1_ref, w2_ref, b2_ref, g2_ref, be2_ref, wh_ref, bh_ref,
                  out_ref):
    f32 = jnp.float32
    um = um_ref[...]
    im = im_ref[...]
    w1 = w1_ref[...]
    inv1 = g1_ref[...] / jnp.sqrt(1.0 + EPS)
    inv2 = g2_ref[...] / jnp.sqrt(1.0 + EPS)
    h = lax.dot_general(um, w1[:, :EMB], (((1,), (1,)), ((), ())),
                        preferred_element_type=f32)
    h += lax.dot_general(im, w1[:, EMB:], (((1,), (1,)), ((), ())),
                         preferred_element_type=f32)
    h = (h + b1_ref[...]) * inv1 + be1_ref[...]
    h = jnp.maximum(h, 0.0)
    h = lax.dot_general(h, w2_ref[...], (((1,), (1,)), ((), ())),
                        preferred_element_type=f32)
    h = (h + b2_ref[...]) * inv2 + be2_ref[...]
    h = jnp.maximum(h, 0.0)
    wh = wh_ref[...]
    gmf = ug_ref[...] * ig_ref[...]
    out = jnp.sum(gmf * wh[:, :EMB], axis=1) + jnp.sum(h * wh[:, EMB:], axis=1)
    out = out + bh_ref[0, 0]
    out_ref[...] = jnp.clip(out, -2.0, 2.0)


def _tc_tail(ug, ig, um, im, W1, b1, g1, be1, W2, b2, g2, be2, Wh, bh):
    blk = 2048
    grid = (BATCH // blk,)
    rows = pl.BlockSpec((blk, EMB), lambda i: (i, 0))
    full = lambda a: pl.BlockSpec(a.shape, lambda i: (0,) * a.ndim)
    args = (W1, b1, g1, be1, W2, b2, g2, be2, Wh, bh)
    return pl.pallas_call(
        _tc_tail_body,
        grid=grid,
        in_specs=[rows, rows, rows, rows] + [full(a) for a in args],
        out_specs=pl.BlockSpec((blk,), lambda i: (i,)),
        out_shape=jax.ShapeDtypeStruct((BATCH,), jnp.float32),
    )(ug, ig, um, im, *args)


def kernel(x, Ug, Ig, Um, Im, W1, b1, g1, be1, W2, b2, g2, be2, Wh, bh):
    xi = x.astype(jnp.int32)
    uidx = xi[:, 0]
    iidx = xi[:, 1]
    ug, ig, um, im = _sc_gather(uidx, iidx, Ug, Ig, Um, Im)
    return _tc_tail(ug, ig, um, im,
                    W1, b1.reshape(1, -1), g1.reshape(1, -1),
                    be1.reshape(1, -1), W2, b2.reshape(1, -1),
                    g2.reshape(1, -1), be2.reshape(1, -1), Wh,
                    bh.reshape(1, -1))


# speedup vs baseline: 1.8199x; 1.8199x over previous
import functools

import jax
import jax.numpy as jnp
from jax import lax
from jax.experimental import pallas as pl
from jax.experimental.pallas import tpu as pltpu
from jax.experimental.pallas import tpu_sc as plsc

BATCH = 16384
EMB = 32
EPS = 1e-5

NC = 2
NS = 16
NW = NC * NS
BPW = BATCH // NW
CHUNK = 128
NCHUNK = BPW // CHUNK


def _sc_gather_body(uidx, iidx, Ug, Ig, Um, Im,
                    o_ug, o_ig, o_um, o_im,
                    vu, vi, st_ug, st_ig, st_um, st_im, sem):
    wid = lax.axis_index("s") * NC + lax.axis_index("c")
    base = wid * BPW
    pltpu.sync_copy(uidx.at[pl.ds(base, BPW)], vu)
    pltpu.sync_copy(iidx.at[pl.ds(base, BPW)], vi)

    def chunk(c, _):
        off = c * CHUNK

        def group(g, _):
            o2 = off + g * 16
            uvec = vu[pl.ds(o2, 16)]
            ivec = vi[pl.ds(o2, 16)]
            for k in range(16):
                u = uvec[k]
                t = ivec[k]
                r = g * 16 + k
                pltpu.make_async_copy(Ug.at[u], st_ug.at[r], sem).start()
                pltpu.make_async_copy(Um.at[u], st_um.at[r], sem).start()
                pltpu.make_async_copy(Ig.at[t], st_ig.at[r], sem).start()
                pltpu.make_async_copy(Im.at[t], st_im.at[r], sem).start()
            return ()

        lax.fori_loop(0, CHUNK // 16, group, (), unroll=False)
        for st in (st_ug, st_um, st_ig, st_im):
            pltpu.make_async_copy(Ug.at[pl.ds(0, CHUNK)], st, sem).wait()
        dst = pl.ds(base + off, CHUNK)
        pltpu.sync_copy(st_ug, o_ug.at[dst])
        pltpu.sync_copy(st_um, o_um.at[dst])
        pltpu.sync_copy(st_ig, o_ig.at[dst])
        pltpu.sync_copy(st_im, o_im.at[dst])
        return ()

    lax.fori_loop(0, NCHUNK, chunk, (), unroll=False)


def _sc_gather(uidx, iidx, Ug, Ig, Um, Im):
    mesh = plsc.VectorSubcoreMesh(core_axis_name="c", subcore_axis_name="s",
                                  num_cores=NC, num_subcores=NS)
    row = jax.ShapeDtypeStruct((BATCH, EMB), jnp.float32)
    f = pl.kernel(
        _sc_gather_body,
        out_type=(row, row, row, row),
        mesh=mesh,
        scratch_types=[
            pltpu.VMEM((BPW,), jnp.int32),
            pltpu.VMEM((BPW,), jnp.int32),
            pltpu.VMEM((CHUNK, EMB), jnp.float32),
            pltpu.VMEM((CHUNK, EMB), jnp.float32),
            pltpu.VMEM((CHUNK, EMB), jnp.float32),
            pltpu.VMEM((CHUNK, EMB), jnp.float32),
            pltpu.SemaphoreType.DMA,
        ],
    )
    return f(uidx, iidx, Ug, Ig, Um, Im)


def _tc_tail_body(ug_ref, ig_ref, um_ref, im_ref, w1_ref, b1_ref, g1_ref,
                  be1_ref, w2_ref, b2_ref, g2_ref, be2_ref, wh_ref, bh_ref,
                  out_ref):
    f32 = jnp.float32
    um = um_ref[...]
    im = im_ref[...]
    w1 = w1_ref[...]
    inv1 = g1_ref[...] / jnp.sqrt(1.0 + EPS)
    inv2 = g2_ref[...] / jnp.sqrt(1.0 + EPS)
    h = lax.dot_general(um, w1[:, :EMB], (((1,), (1,)), ((), ())),
                        preferred_element_type=f32)
    h += lax.dot_general(im, w1[:, EMB:], (((1,), (1,)), ((), ())),
                         preferred_element_type=f32)
    h = (h + b1_ref[...]) * inv1 + be1_ref[...]
    h = jnp.maximum(h, 0.0)
    h = lax.dot_general(h, w2_ref[...], (((1,), (1,)), ((), ())),
                        preferred_element_type=f32)
    h = (h + b2_ref[...]) * inv2 + be2_ref[...]
    h = jnp.maximum(h, 0.0)
    wh = wh_ref[...]
    gmf = ug_ref[...] * ig_ref[...]
    out = jnp.sum(gmf * wh[:, :EMB], axis=1) + jnp.sum(h * wh[:, EMB:], axis=1)
    out = out + bh_ref[0, 0]
    out_ref[...] = jnp.clip(out, -2.0, 2.0)


def _tc_tail(ug, ig, um, im, W1, b1, g1, be1, W2, b2, g2, be2, Wh, bh):
    blk = 2048
    grid = (BATCH // blk,)
    rows = pl.BlockSpec((blk, EMB), lambda i: (i, 0))
    full = lambda a: pl.BlockSpec(a.shape, lambda i: (0,) * a.ndim)
    args = (W1, b1, g1, be1, W2, b2, g2, be2, Wh, bh)
    return pl.pallas_call(
        _tc_tail_body,
        grid=grid,
        in_specs=[rows, rows, rows, rows] + [full(a) for a in args],
        out_specs=pl.BlockSpec((blk,), lambda i: (i,)),
        out_shape=jax.ShapeDtypeStruct((BATCH,), jnp.float32),
    )(ug, ig, um, im, *args)


def kernel(x, Ug, Ig, Um, Im, W1, b1, g1, be1, W2, b2, g2, be2, Wh, bh):
    xi = x.astype(jnp.int32)
    uidx = xi[:, 0]
    iidx = xi[:, 1]
    ug, ig, um, im = _sc_gather(uidx, iidx, Ug, Ig, Um, Im)
    return _tc_tail(ug, ig, um, im,
                    W1, b1.reshape(1, -1), g1.reshape(1, -1),
                    be1.reshape(1, -1), W2, b2.reshape(1, -1),
                    g2.reshape(1, -1), be2.reshape(1, -1), Wh,
                    bh.reshape(1, -1))


# speedup vs baseline: 1.8202x; 1.0002x over previous
import functools

import jax
import jax.numpy as jnp
from jax import lax
from jax.experimental import pallas as pl
from jax.experimental.pallas import tpu as pltpu
from jax.experimental.pallas import tpu_sc as plsc

BATCH = 16384
EMB = 32
EPS = 1e-5

NC = 2
NS = 16
NW = NC * NS
BPW = BATCH // NW
CHUNK = 128
NCHUNK = BPW // CHUNK


def _sc_gather_body(uidx, iidx, Ug, Ig, Um, Im,
                    o_ug, o_ig, o_um, o_im,
                    vu, vi, st_ug, st_ig, st_um, st_im, sem):
    wid = lax.axis_index("s") * NC + lax.axis_index("c")
    base = wid * BPW
    pltpu.sync_copy(uidx.at[pl.ds(base, BPW)], vu)
    pltpu.sync_copy(iidx.at[pl.ds(base, BPW)], vi)

    def chunk(c, _):
        off = c * CHUNK

        @plsc.parallel_loop(0, CHUNK // 16, step=1, unroll=2)
        def group(g):
            o2 = off + g * 16
            uvec = vu[pl.ds(o2, 16)]
            ivec = vi[pl.ds(o2, 16)]
            for k in range(16):
                u = uvec[k]
                t = ivec[k]
                r = g * 16 + k
                pltpu.make_async_copy(Ug.at[u], st_ug.at[r], sem).start()
                pltpu.make_async_copy(Um.at[u], st_um.at[r], sem).start()
                pltpu.make_async_copy(Ig.at[t], st_ig.at[r], sem).start()
                pltpu.make_async_copy(Im.at[t], st_im.at[r], sem).start()
        for st in (st_ug, st_um, st_ig, st_im):
            pltpu.make_async_copy(Ug.at[pl.ds(0, CHUNK)], st, sem).wait()
        dst = pl.ds(base + off, CHUNK)
        pltpu.sync_copy(st_ug, o_ug.at[dst])
        pltpu.sync_copy(st_um, o_um.at[dst])
        pltpu.sync_copy(st_ig, o_ig.at[dst])
        pltpu.sync_copy(st_im, o_im.at[dst])
        return ()

    lax.fori_loop(0, NCHUNK, chunk, (), unroll=False)


def _sc_gather(uidx, iidx, Ug, Ig, Um, Im):
    mesh = plsc.VectorSubcoreMesh(core_axis_name="c", subcore_axis_name="s",
                                  num_cores=NC, num_subcores=NS)
    row = jax.ShapeDtypeStruct((BATCH, EMB), jnp.float32)
    f = pl.kernel(
        _sc_gather_body,
        out_type=(row, row, row, row),
        mesh=mesh,
        scratch_types=[
            pltpu.VMEM((BPW,), jnp.int32),
            pltpu.VMEM((BPW,), jnp.int32),
            pltpu.VMEM((CHUNK, EMB), jnp.float32),
            pltpu.VMEM((CHUNK, EMB), jnp.float32),
            pltpu.VMEM((CHUNK, EMB), jnp.float32),
            pltpu.VMEM((CHUNK, EMB), jnp.float32),
            pltpu.SemaphoreType.DMA,
        ],
    )
    return f(uidx, iidx, Ug, Ig, Um, Im)


def _tc_tail_body(ug_ref, ig_ref, um_ref, im_ref, w1_ref, b1_ref, g1_ref,
                  be1_ref, w2_ref, b2_ref, g2_ref, be2_ref, wh_ref, bh_ref,
                  out_ref):
    f32 = jnp.float32
    um = um_ref[...]
    im = im_ref[...]
    w1 = w1_ref[...]
    inv1 = g1_ref[...] / jnp.sqrt(1.0 + EPS)
    inv2 = g2_ref[...] / jnp.sqrt(1.0 + EPS)
    h = lax.dot_general(um, w1[:, :EMB], (((1,), (1,)), ((), ())),
                        preferred_element_type=f32)
    h += lax.dot_general(im, w1[:, EMB:], (((1,), (1,)), ((), ())),
                         preferred_element_type=f32)
    h = (h + b1_ref[...]) * inv1 + be1_ref[...]
    h = jnp.maximum(h, 0.0)
    h = lax.dot_general(h, w2_ref[...], (((1,), (1,)), ((), ())),
                        preferred_element_type=f32)
    h = (h + b2_ref[...]) * inv2 + be2_ref[...]
    h = jnp.maximum(h, 0.0)
    wh = wh_ref[...]
    gmf = ug_ref[...] * ig_ref[...]
    out = jnp.sum(gmf * wh[:, :EMB], axis=1) + jnp.sum(h * wh[:, EMB:], axis=1)
    out = out + bh_ref[0, 0]
    out_ref[...] = jnp.clip(out, -2.0, 2.0)


def _tc_tail(ug, ig, um, im, W1, b1, g1, be1, W2, b2, g2, be2, Wh, bh):
    blk = 2048
    grid = (BATCH // blk,)
    rows = pl.BlockSpec((blk, EMB), lambda i: (i, 0))
    full = lambda a: pl.BlockSpec(a.shape, lambda i: (0,) * a.ndim)
    args = (W1, b1, g1, be1, W2, b2, g2, be2, Wh, bh)
    return pl.pallas_call(
        _tc_tail_body,
        grid=grid,
        in_specs=[rows, rows, rows, rows] + [full(a) for a in args],
        out_specs=pl.BlockSpec((blk,), lambda i: (i,)),
        out_shape=jax.ShapeDtypeStruct((BATCH,), jnp.float32),
    )(ug, ig, um, im, *args)


def kernel(x, Ug, Ig, Um, Im, W1, b1, g1, be1, W2, b2, g2, be2, Wh, bh):
    xi = x.astype(jnp.int32)
    uidx = xi[:, 0]
    iidx = xi[:, 1]
    ug, ig, um, im = _sc_gather(uidx, iidx, Ug, Ig, Um, Im)
    return _tc_tail(ug, ig, um, im,
                    W1, b1.reshape(1, -1), g1.reshape(1, -1),
                    be1.reshape(1, -1), W2, b2.reshape(1, -1),
                    g2.reshape(1, -1), be2.reshape(1, -1), Wh,
                    bh.reshape(1, -1))
